# baseline (speedup 1.0000x reference)
import functools

import jax
import jax.numpy as jnp
from jax import lax
from jax.experimental import pallas as pl
from jax.experimental.pallas import tpu as pltpu
from jax.experimental.pallas import tpu_sc as plsc

NUM_ENTITIES = 1000000
EMBED_DIM = 64
BATCH = 16384
LANES = 128
NBUF = 8
GRP = 16


@functools.cache
def _build_gather():
    info = plsc.get_sparse_core_info()
    nc, ns = info.num_cores, info.num_subcores
    nw = nc * ns
    b_per_w = BATCH // nw
    n_grp = b_per_w // GRP

    mesh = plsc.VectorSubcoreMesh(core_axis_name="c", subcore_axis_name="s")

    @functools.partial(
        pl.kernel,
        mesh=mesh,
        out_type=jax.ShapeDtypeStruct((EMBED_DIM, BATCH), jnp.float32),
        scratch_types=[
            pltpu.VMEM((b_per_w,), jnp.int32),
            pltpu.VMEM((EMBED_DIM, b_per_w), jnp.float32),
        ]
        + [pltpu.VMEM((EMBED_DIM, LANES), jnp.float32) for _ in range(NBUF)]
        + [pltpu.SemaphoreType.DMA for _ in range(2 * NBUF)],
        compiler_params=pltpu.CompilerParams(needs_layout_passes=False),
    )
    def gather(idx_hbm, table_t_hbm, out_hbm, idx_v, staging, *bufs_sems):
        bufs = bufs_sems[:NBUF]
        sems = bufs_sems[NBUF:]
        half = EMBED_DIM // 2
        wid = lax.axis_index("s") * nc + lax.axis_index("c")
        base = wid * b_per_w
        pltpu.sync_copy(idx_hbm.at[pl.ds(base, b_per_w)], idx_v)

        lane_iota = lax.iota(jnp.int32, 16)

        def fire(r, b):
            t = pl.multiple_of((r >> 7) << 7, LANES)
            for h in range(2):
                pltpu.make_async_copy(
                    table_t_hbm.at[pl.ds(h * half, half), pl.ds(t, LANES)],
                    bufs[b].at[pl.ds(h * half, half), :],
                    sems[2 * b + h],
                ).start()

        idx16_0 = idx_v[pl.ds(0, GRP)]
        for b in range(NBUF):
            fire(idx16_0[b], b)

        def group(g, idx16):
            nxt_off = pl.multiple_of(
                jnp.where(g == n_grp - 1, 0, (g + 1) * GRP), GRP
            )
            idx16_nxt = idx_v[pl.ds(nxt_off, GRP)]
            i0 = g * GRP
            for b in range(GRP):
                s = b % NBUF
                for h in range(2):
                    pltpu.make_async_copy(
                        table_t_hbm.at[pl.ds(h * half, half), pl.ds(0, LANES)],
                        bufs[s].at[pl.ds(h * half, half), :],
                        sems[2 * s + h],
                    ).wait()
                r = idx16[b]
                lane = jnp.full((16,), r & (LANES - 1), jnp.int32)
                col_idx = jnp.full((16,), i0 + b, jnp.int32)
                for k in range(EMBED_DIM // 16):
                    feat = lane_iota + (16 * k)
                    v = plsc.load_gather(bufs[s], [feat, lane])
                    plsc.store_scatter(staging, [feat, col_idx], v)
                if b + NBUF < GRP:
                    fire(idx16[b + NBUF], s)
                else:

                    @pl.when(g < n_grp - 1)
                    def _():
                        fire(idx16_nxt[b + NBUF - GRP], s)

            return idx16_nxt

        lax.fori_loop(0, n_grp, group, idx16_0)
        pltpu.sync_copy(staging, out_hbm.at[:, pl.ds(base, b_per_w)])

    return gather


def kernel(entities, entity_embeddings):
    out_t = _build_gather()(entities.astype(jnp.int32), entity_embeddings.T)
    return out_t.T


# speedup vs baseline: 1.1100x; 1.1100x over previous
import functools

import jax
import jax.numpy as jnp
from jax import lax
from jax.experimental import pallas as pl
from jax.experimental.pallas import tpu as pltpu
from jax.experimental.pallas import tpu_sc as plsc

NUM_ENTITIES = 1000000
EMBED_DIM = 64
BATCH = 16384
LANES = 128
NBUF = 8
GRP = 16


@functools.cache
def _build_gather():
    info = plsc.get_sparse_core_info()
    nc, ns = info.num_cores, info.num_subcores
    nw = nc * ns
    b_per_w = BATCH // nw
    n_grp = b_per_w // GRP

    mesh = plsc.VectorSubcoreMesh(core_axis_name="c", subcore_axis_name="s")

    @functools.partial(
        pl.kernel,
        mesh=mesh,
        out_type=jax.ShapeDtypeStruct((EMBED_DIM, BATCH), jnp.float32),
        scratch_types=[
            pltpu.VMEM((b_per_w,), jnp.int32),
            pltpu.VMEM((EMBED_DIM, b_per_w), jnp.float32),
        ]
        + [pltpu.VMEM((EMBED_DIM, LANES), jnp.float32) for _ in range(NBUF)]
        + [pltpu.SemaphoreType.DMA for _ in range(NBUF)],
        compiler_params=pltpu.CompilerParams(needs_layout_passes=False),
    )
    def gather(idx_hbm, table_t_hbm, out_hbm, idx_v, staging, *bufs_sems):
        bufs = bufs_sems[:NBUF]
        sems = bufs_sems[NBUF:]
        wid = lax.axis_index("s") * nc + lax.axis_index("c")
        base = wid * b_per_w
        pltpu.sync_copy(idx_hbm.at[pl.ds(base, b_per_w)], idx_v)

        lane_iota = lax.iota(jnp.int32, 16)

        def fire(r, b):
            t = pl.multiple_of((r >> 7) << 7, LANES)
            pltpu.make_async_copy(
                table_t_hbm.at[:, pl.ds(t, LANES)], bufs[b], sems[b]
            ).start()

        idx16_0 = idx_v[pl.ds(0, GRP)]
        for b in range(NBUF):
            fire(idx16_0[b], b)

        def group(g, idx16):
            nxt_off = pl.multiple_of(
                jnp.where(g == n_grp - 1, 0, (g + 1) * GRP), GRP
            )
            idx16_nxt = idx_v[pl.ds(nxt_off, GRP)]
            i0 = g * GRP
            for b in range(GRP):
                s = b % NBUF
                pltpu.make_async_copy(
                    table_t_hbm.at[:, pl.ds(0, LANES)], bufs[s], sems[s]
                ).wait()
                r = idx16[b]
                lane = jnp.full((16,), r & (LANES - 1), jnp.int32)
                col_idx = jnp.full((16,), i0 + b, jnp.int32)
                for k in range(EMBED_DIM // 16):
                    feat = lane_iota + (16 * k)
                    v = plsc.load_gather(bufs[s], [feat, lane])
                    plsc.store_scatter(staging, [feat, col_idx], v)
                if b + NBUF < GRP:
                    fire(idx16[b + NBUF], s)
                else:

                    @pl.when(g < n_grp - 1)
                    def _():
                        fire(idx16_nxt[b + NBUF - GRP], s)

            return idx16_nxt

        lax.fori_loop(0, n_grp, group, idx16_0)
        pltpu.sync_copy(staging, out_hbm.at[:, pl.ds(base, b_per_w)])

    return gather


def kernel(entities, entity_embeddings):
    out_t = _build_gather()(entities.astype(jnp.int32), entity_embeddings.T)
    return out_t.T


# speedup vs baseline: 1.7038x; 1.5349x over previous
import functools

import jax
import jax.numpy as jnp
from jax import lax
from jax.experimental import pallas as pl
from jax.experimental.pallas import tpu as pltpu
from jax.experimental.pallas import tpu_sc as plsc

NUM_ENTITIES = 1000000
EMBED_DIM = 64
BATCH = 16384
LANES = 128
NT = (NUM_ENTITIES + LANES - 1) // LANES
NBUF = 6
OUT_PAD = 16
SROWS = 32


@functools.cache
def _build_gather():
    info = plsc.get_sparse_core_info()
    nc, ns = info.num_cores, info.num_subcores
    nw = nc * ns
    tpw = (NT + nw - 1) // nw
    cap = BATCH + tpw * 15 + 16

    mesh = plsc.VectorSubcoreMesh(core_axis_name="c", subcore_axis_name="s")

    @functools.partial(
        pl.kernel,
        mesh=mesh,
        out_type=jax.ShapeDtypeStruct((BATCH + OUT_PAD, LANES), jnp.float32),
        scratch_types=[
            pltpu.VMEM((BATCH,), jnp.int32),
            pltpu.VMEM((BATCH,), jnp.int32),
            pltpu.VMEM((cap,), jnp.int32),
            pltpu.VMEM((cap,), jnp.int32),
            pltpu.VMEM((NBUF, EMBED_DIM, LANES), jnp.float32),
            pltpu.VMEM((SROWS, LANES), jnp.float32),
            pltpu.VMEM((SROWS,), jnp.int32),
            pltpu.SMEM((tpw,), jnp.int32),
            pltpu.SMEM((tpw,), jnp.int32),
            pltpu.SMEM((tpw,), jnp.int32),
            pltpu.SMEM((NBUF,), jnp.int32),
            pltpu.SemaphoreType.DMA((NBUF,)),
            pltpu.SemaphoreType.DMA,
        ],
        compiler_params=pltpu.CompilerParams(needs_layout_passes=False),
    )
    def gather(
        idx_hbm,
        table_t_hbm,
        out_hbm,
        idx_all,
        mpos,
        gid,
        gpos,
        bufs,
        stag,
        stagpos,
        hist,
        goff,
        gcur,
        pend,
        fsem,
        osem,
    ):
        wid = lax.axis_index("s") * nc + lax.axis_index("c")
        lo = wid * tpw
        hi = jnp.minimum(lo + tpw, NT)
        pltpu.sync_copy(idx_hbm, idx_all)

        iota = lax.iota(jnp.int32, 16)
        lane0 = iota < 1

        def p1(g, cnt):
            v = idx_all[pl.ds(pl.multiple_of(g * 16, 8), 16)]
            t = v >> 7
            m = (t >= lo) & (t < hi)
            s = plsc.cumsum(jnp.where(m, 1, 0))
            dst = cnt + s - 1
            plsc.store_scatter(idx_all, [dst], v, mask=m)
            plsc.store_scatter(mpos, [dst], iota + g * 16, mask=m)
            return cnt + s[15]

        cnt = lax.fori_loop(0, BATCH // 16, p1, jnp.int32(0))
        nchunks = (cnt + 15) >> 4

        def p2_clear(c, carry):
            hist[c] = 0
            return carry

        lax.fori_loop(0, tpw, p2_clear, None)

        def p2(g, carry):
            v = idx_all[pl.ds(pl.multiple_of(g * 16, 8), 16)]
            for b in range(16):

                @pl.when(g * 16 + b < cnt)
                def _():
                    t = (v[b] >> 7) - lo
                    hist[t] = hist[t] + 1

            return carry

        lax.fori_loop(0, nchunks, p2, None)

        def p3(c, run):
            goff[c] = run
            gcur[c] = run
            return run + ((hist[c] + 15) & ~15)

        lax.fori_loop(0, tpw, p3, jnp.int32(0))

        def p4(g, carry):
            v = idx_all[pl.ds(pl.multiple_of(g * 16, 8), 16)]
            p = mpos[pl.ds(pl.multiple_of(g * 16, 8), 16)]
            for b in range(16):

                @pl.when(g * 16 + b < cnt)
                def _():
                    t = (v[b] >> 7) - lo
                    slot = gcur[t]
                    gcur[t] = slot + 1
                    sv = jnp.full((16,), slot, jnp.int32)
                    plsc.store_scatter(
                        gid, [sv], jnp.full((16,), v[b], jnp.int32), mask=lane0
                    )
                    plsc.store_scatter(
                        gpos, [sv], jnp.full((16,), p[b], jnp.int32), mask=lane0
                    )

            return carry

        lax.fori_loop(0, nchunks, p4, None)

        def wait_one_flush():
            pltpu.make_async_copy(
                table_t_hbm.at[pl.ds(0, 16), pl.ds(0, LANES)],
                stag.at[pl.ds(0, 16), :],
                osem,
            ).wait()

        def flush(blk, started):
            @pl.when(started >= 1)
            def _():
                wait_one_flush()

            pltpu.make_async_copy(
                stag.at[pl.ds(blk, 16), :],
                out_hbm.at[stagpos.at[pl.ds(blk, 16)]],
                osem,
            ).start()
            return started + 1

        def fire(c, s):
            t = pl.multiple_of((lo + c) << 7, LANES)
            pltpu.make_async_copy(
                table_t_hbm.at[:, pl.ds(t, LANES)], bufs.at[s], fsem.at[s]
            ).start()

        def extract(c, s, ocur, started):
            n = hist[c]
            o = goff[c]
            svec = jnp.full((16,), s, jnp.int32)

            def chunk(j, carry):
                ocur, started = carry
                og = pl.multiple_of(o + j * 16, 16)
                rv = gid[pl.ds(og, 16)]
                pv = gpos[pl.ds(og, 16)]
                valid = iota < (n - j * 16)
                lanes = rv & (LANES - 1)
                nv = plsc.cumsum(jnp.where(valid, 1, 0))
                rows = (ocur + nv - 1) & (SROWS - 1)
                for ff in range(EMBED_DIM):
                    fv = jnp.full((16,), ff, jnp.int32)
                    v = plsc.load_gather(bufs, [svec, fv, lanes], mask=valid)
                    plsc.store_scatter(stag, [rows, fv], v, mask=valid)
                plsc.store_scatter(stagpos, [rows], pv, mask=valid)
                ocur2 = ocur + nv[15]
                crossed = (ocur2 >> 4) != (ocur >> 4)
                started2 = lax.cond(
                    crossed,
                    lambda st: flush(
                        pl.multiple_of((ocur >> 4 << 4) & (SROWS - 1), 16), st
                    ),
                    lambda st: st,
                    started,
                )
                return ocur2, started2

            return lax.fori_loop(0, (n + 15) >> 4, chunk, (ocur, started))

        def main(c, carry):
            f, ocur, started = carry

            def do(carry):
                f, ocur, started = carry
                s = lax.rem(f, NBUF)

                def drain(carry):
                    ocur, started = carry
                    cprev = pend[s]
                    pltpu.make_async_copy(
                        table_t_hbm.at[:, pl.ds(0, LANES)], bufs.at[s], fsem.at[s]
                    ).wait()
                    return extract(cprev, s, ocur, started)

                ocur, started = lax.cond(
                    f >= NBUF, drain, lambda x: x, (ocur, started)
                )
                fire(c, s)
                pend[s] = c
                return f + 1, ocur, started

            return lax.cond(hist[c] > 0, do, lambda x: x, (f, ocur, started))

        f, ocur, started = lax.fori_loop(
            0, tpw, main, (jnp.int32(0), jnp.int32(0), jnp.int32(0))
        )

        def tail(k, carry):
            f0, ocur, started = carry

            def do(carry):
                f0, ocur, started = carry
                s = lax.rem(f0, NBUF)
                cprev = pend[s]
                pltpu.make_async_copy(
                    table_t_hbm.at[:, pl.ds(0, LANES)], bufs.at[s], fsem.at[s]
                ).wait()
                ocur, started = extract(cprev, s, ocur, started)
                return f0 + 1, ocur, started

            return lax.cond(f0 < f, do, lambda x: x, (f0, ocur, started))

        start0 = jnp.maximum(f - NBUF, 0)
        _, ocur, started = lax.fori_loop(0, NBUF, tail, (start0, ocur, started))

        def ragged(st):
            blk = pl.multiple_of((ocur >> 4 << 4) & (SROWS - 1), 16)
            nvalid = ocur & 15
            pv = stagpos[pl.ds(blk, 16)]
            pv2 = jnp.where(iota < nvalid, pv, BATCH + iota)
            plsc.store_scatter(stagpos, [blk + iota], pv2)
            return flush(blk, st)

        started = lax.cond((ocur & 15) > 0, ragged, lambda st: st, started)

        @pl.when(started >= 1)
        def _():
            wait_one_flush()

    return gather


def kernel(entities, entity_embeddings):
    out = _build_gather()(entities.astype(jnp.int32), entity_embeddings.T)
    return out[:BATCH, :EMBED_DIM]
